# Initial kernel scaffold; baseline (speedup 1.0000x reference)
import jax
import jax.numpy as jnp
from jax.experimental import pallas as pl


def kernel(x, edge_index, W1, b1, W2, b2):
    raise NotImplementedError("write your pallas kernel here")


# speedup vs baseline: 14.3307x; 14.3307x over previous
import functools

import jax
import jax.numpy as jnp
from jax import lax
from jax.experimental import pallas as pl
from jax.experimental.pallas import tpu as pltpu
from jax.experimental.pallas import tpu_sc as plsc

N_NODES = 10000
N_EDGES = 320000
D_IN = 128
D_HID = 128
D_OUT = 16

NC = 2
NS = 16
NW = NC * NS
L = 16

B = 128
G = 80
PH = 2
GP = G // PH
E_PAD = NW * G * B
N_TRASH = 112
N_ACC = N_NODES + N_TRASH
ROWS_PER_TILE = N_ACC // NS


def _sc_edge_agg(d):
    mesh = plsc.VectorSubcoreMesh(core_axis_name="c", subcore_axis_name="s")

    @functools.partial(
        pl.kernel,
        out_type=jax.ShapeDtypeStruct((NC, N_ACC, d), jnp.float32),
        mesh=mesh,
        scratch_types=[
            pltpu.VMEM((GP, B), jnp.int32),
            pltpu.VMEM((GP, B), jnp.int32),
            pltpu.VMEM((2, B, d), jnp.float32),
            pltpu.VMEM_SHARED((N_ACC, d), jnp.float32),
            pltpu.SemaphoreType.DMA,
            pltpu.SemaphoreType.DMA,
        ],
        compiler_params=pltpu.CompilerParams(use_tc_tiling_on_sc=(d % 128 == 0)),
    )
    def agg(srcp_hbm, dstp_hbm, feat_hbm, out_hbm,
            srcv, dstv, rows, acc, sem0, sem1):
        cid = lax.axis_index("c")
        sid = lax.axis_index("s")
        wid = cid * NS + sid

        zero = jnp.zeros((L,), jnp.float32)

        def zrow(r, carry):
            for c in range(d // L):
                rows[0, r, pl.ds(c * L, L)] = zero
            return carry

        lax.fori_loop(0, B, zrow, 0)
        zbase = sid * ROWS_PER_TILE
        for z in range(4):
            pltpu.sync_copy(rows.at[0], acc.at[pl.ds(zbase + z * B, B)])
        pltpu.sync_copy(rows.at[0, pl.ds(0, ROWS_PER_TILE - 4 * B)],
                        acc.at[pl.ds(zbase + 4 * B, ROWS_PER_TILE - 4 * B)])
        plsc.subcore_barrier()

        for ph in range(PH):
            pltpu.sync_copy(srcp_hbm.at[wid, pl.ds(ph * GP, GP)], srcv)
            pltpu.sync_copy(dstp_hbm.at[wid, pl.ds(ph * GP, GP)], dstv)

            pltpu.async_copy(feat_hbm.at[srcv.at[0]], rows.at[0], sem0)
            pltpu.async_copy(feat_hbm.at[srcv.at[1]], rows.at[1], sem1)

            def body(j, carry):
                g0 = 2 * j
                g1 = g0 + 1
                pltpu.make_async_copy(
                    feat_hbm.at[srcv.at[g0]], rows.at[0], sem0).wait()
                pltpu.sync_copy(rows.at[0], acc.at[dstv.at[g0]], add=True)

                @pl.when(g0 + 2 < GP)
                def _():
                    pltpu.async_copy(
                        feat_hbm.at[srcv.at[g0 + 2]], rows.at[0], sem0)

                pltpu.make_async_copy(
                    feat_hbm.at[srcv.at[g1]], rows.at[1], sem1).wait()
                pltpu.sync_copy(rows.at[1], acc.at[dstv.at[g1]], add=True)

                @pl.when(g1 + 2 < GP)
                def _():
                    pltpu.async_copy(
                        feat_hbm.at[srcv.at[g1 + 2]], rows.at[1], sem1)

                return carry

            lax.fori_loop(0, GP // 2, body, 0)

        plsc.subcore_barrier()

        pltpu.sync_copy(acc.at[pl.ds(zbase, ROWS_PER_TILE)],
                        out_hbm.at[cid, pl.ds(zbase, ROWS_PER_TILE)])

    return agg


def _dense_body(p_ref, w1_ref, b1_ref, w2_ref, o_ref):
    ps = p_ref[0] + p_ref[1]
    h1 = jnp.dot(ps, w1_ref[...], preferred_element_type=jnp.float32)
    h1 = jnp.maximum(h1 + b1_ref[...], 0.0)
    o_ref[...] = jnp.dot(h1, w2_ref[...], preferred_element_type=jnp.float32)


def _final_body(q_ref, b2_ref, o_ref):
    o_ref[...] = q_ref[0, :N_NODES] + q_ref[1, :N_NODES] + b2_ref[...]


def kernel(x, edge_index, W1, b1, W2, b2):
    src = edge_index[0]
    dst = edge_index[1]

    pad = E_PAD - N_EDGES
    k = lax.iota(jnp.int32, pad)
    srcp = jnp.concatenate([src, k % N_NODES]).reshape(NW, G, B)
    dstp = jnp.concatenate([dst, N_NODES + k % N_TRASH]).reshape(NW, G, B)

    p = _sc_edge_agg(D_IN)(srcp, dstp, x)

    rb = N_ACC // 4
    h2 = pl.pallas_call(
        _dense_body,
        grid=(4,),
        in_specs=[
            pl.BlockSpec((NC, rb, D_IN), lambda i: (0, i, 0)),
            pl.BlockSpec((D_IN, D_HID), lambda i: (0, 0)),
            pl.BlockSpec((1, D_HID), lambda i: (0, 0)),
            pl.BlockSpec((D_HID, D_OUT), lambda i: (0, 0)),
        ],
        out_specs=pl.BlockSpec((rb, D_OUT), lambda i: (i, 0)),
        out_shape=jax.ShapeDtypeStruct((N_ACC, D_OUT), jnp.float32),
    )(p, W1, b1.reshape(1, D_HID), W2)

    q = _sc_edge_agg(D_OUT)(srcp, dstp, h2)

    out = pl.pallas_call(
        _final_body,
        in_specs=[
            pl.BlockSpec((NC, N_ACC, D_OUT), lambda: (0, 0, 0)),
            pl.BlockSpec((1, D_OUT), lambda: (0, 0)),
        ],
        out_specs=pl.BlockSpec((N_NODES, D_OUT), lambda: (0, 0)),
        out_shape=jax.ShapeDtypeStruct((N_NODES, D_OUT), jnp.float32),
    )(q, b2.reshape(1, D_OUT))
    return out
